# baseline (speedup 1.0000x reference)
import functools

import jax
import jax.numpy as jnp
from jax import lax
from jax.experimental import pallas as pl
from jax.experimental.pallas import tpu as pltpu
from jax.experimental.pallas import tpu_sc as plsc


def _tc_block(gu_ref, gi_ref, xui_ref, gu_out_ref):
    u = gu_ref[:, :]
    v = gi_ref[:, :]
    gu_out_ref[:, :] = u
    xui_ref[:] = jnp.sum(u * v, axis=1)


def _make_tc_body(B, D, blk):
    def tc_body(gu_hbm, gi_hbm, xui_hbm, gu_out_hbm):
        pltpu.emit_pipeline(
            _tc_block,
            grid=(B // blk,),
            in_specs=[
                pl.BlockSpec((blk, D), lambda i: (i, 0)),
                pl.BlockSpec((blk, D), lambda i: (i, 0)),
            ],
            out_specs=[
                pl.BlockSpec((blk,), lambda i: (i,)),
                pl.BlockSpec((blk, D), lambda i: (i, 0)),
            ],
        )(gu_hbm, gi_hbm, xui_hbm, gu_out_hbm)

    return tc_body


def _make_sc_copy(B, D):
    mesh = plsc.VectorSubcoreMesh(core_axis_name="c", subcore_axis_name="s")
    n_workers = mesh.num_cores * mesh.num_subcores
    rows = B // n_workers
    n_chunks = 4
    crows = rows // n_chunks

    @functools.partial(
        pl.kernel,
        mesh=mesh,
        out_type=jax.ShapeDtypeStruct((B, D), jnp.float32),
        scratch_types=(
            [pltpu.VMEM((crows, D), jnp.float32) for _ in range(n_chunks)]
            + [pltpu.SemaphoreType.DMA for _ in range(2 * n_chunks)]
        ),
    )
    def sc_copy(src_hbm, dst_hbm, *scratch):
        bufs = scratch[:n_chunks]
        sems_in = scratch[n_chunks:2 * n_chunks]
        sems_out = scratch[2 * n_chunks:]
        wid = lax.axis_index("s") * mesh.num_cores + lax.axis_index("c")
        base = wid * rows
        h_in = [
            pltpu.async_copy(src_hbm.at[pl.ds(base + k * crows, crows)],
                             bufs[k], sems_in[k])
            for k in range(n_chunks)
        ]
        h_out = []
        for k in range(n_chunks):
            h_in[k].wait()
            h_out.append(
                pltpu.async_copy(bufs[k],
                                 dst_hbm.at[pl.ds(base + k * crows, crows)],
                                 sems_out[k]))
        for k in range(n_chunks):
            h_out[k].wait()

    return sc_copy


def kernel(gu, gi):
    B, D = gu.shape
    blk = 2048
    gu = pltpu.with_memory_space_constraint(gu, pltpu.MemorySpace.HBM)
    gi = pltpu.with_memory_space_constraint(gi, pltpu.MemorySpace.HBM)
    xui, gu_o = pl.pallas_call(
        _make_tc_body(B, D, blk),
        in_specs=[
            pl.BlockSpec(memory_space=pltpu.MemorySpace.HBM),
            pl.BlockSpec(memory_space=pltpu.MemorySpace.HBM),
        ],
        out_specs=[
            pl.BlockSpec(memory_space=pltpu.MemorySpace.HBM),
            pl.BlockSpec(memory_space=pltpu.MemorySpace.HBM),
        ],
        out_shape=[
            jax.ShapeDtypeStruct((B,), jnp.float32),
            jax.ShapeDtypeStruct((B, D), jnp.float32),
        ],
    )(gu, gi)
    gi_o = _make_sc_copy(B, D)(gi)
    return (xui, gu_o, gi_o)


# speedup vs baseline: 2.2966x; 2.2966x over previous
import functools

import jax
import jax.numpy as jnp
from jax import lax
from jax.experimental import pallas as pl
from jax.experimental.pallas import tpu as pltpu
from jax.experimental.pallas import tpu_sc as plsc


def _tc_block(gu_ref, gi_ref, xui_ref, gu_out_ref, gi_out_ref):
    u = gu_ref[:, :]
    v = gi_ref[:, :]
    gu_out_ref[:, :] = u
    gi_out_ref[:, :] = v
    xui_ref[:] = jnp.sum(u * v, axis=1)


def _make_tc_body(B, D, blk):
    def tc_body(gu_hbm, gi_hbm, xui_hbm, gu_out_hbm, gi_out_hbm):
        pltpu.emit_pipeline(
            _tc_block,
            grid=(B // blk,),
            in_specs=[
                pl.BlockSpec((blk, D), lambda i: (i, 0)),
                pl.BlockSpec((blk, D), lambda i: (i, 0)),
            ],
            out_specs=[
                pl.BlockSpec((blk,), lambda i: (i,)),
                pl.BlockSpec((blk, D), lambda i: (i, 0)),
                pl.BlockSpec((blk, D), lambda i: (i, 0)),
            ],
        )(gu_hbm, gi_hbm, xui_hbm, gu_out_hbm, gi_out_hbm)

    return tc_body


def kernel(gu, gi):
    B, D = gu.shape
    blk = 2048
    gu = pltpu.with_memory_space_constraint(gu, pltpu.MemorySpace.HBM)
    gi = pltpu.with_memory_space_constraint(gi, pltpu.MemorySpace.HBM)
    xui, gu_o, gi_o = pl.pallas_call(
        _make_tc_body(B, D, blk),
        in_specs=[
            pl.BlockSpec(memory_space=pltpu.MemorySpace.HBM),
            pl.BlockSpec(memory_space=pltpu.MemorySpace.HBM),
        ],
        out_specs=[
            pl.BlockSpec(memory_space=pltpu.MemorySpace.HBM),
            pl.BlockSpec(memory_space=pltpu.MemorySpace.HBM),
            pl.BlockSpec(memory_space=pltpu.MemorySpace.HBM),
        ],
        out_shape=[
            jax.ShapeDtypeStruct((B,), jnp.float32),
            jax.ShapeDtypeStruct((B, D), jnp.float32),
            jax.ShapeDtypeStruct((B, D), jnp.float32),
        ],
    )(gu, gi)
    return (xui, gu_o, gi_o)
